# baseline (speedup 1.0000x reference)
import jax
import jax.numpy as jnp
from jax.experimental import pallas as pl
from jax.experimental.pallas import tpu as pltpu

_D = 768
_ROWS = 64
_NB0, _NB1, _NB2 = 16, 4, 1
_NTOT = _NB0 + _NB1 + _NB2


def _body(f0_ref, f1_ref, f2_ref, se_ref, pe_ref, out_ref):
    j = pl.program_id(0)
    pe = pe_ref[...]

    @pl.when(j < _NB0)
    def _():
        out_ref[...] = f0_ref[...] + (se_ref[0, :][None, None, :] + pe)

    @pl.when(jnp.logical_and(j >= _NB0, j < _NB0 + _NB1))
    def _():
        out_ref[...] = f1_ref[...] + (se_ref[1, :][None, None, :] + pe)

    @pl.when(j >= _NB0 + _NB1)
    def _():
        out_ref[...] = f2_ref[...] + (se_ref[2, :][None, None, :] + pe)


def _scale_of(j):
    return jnp.where(j < _NB0, 0, jnp.where(j < _NB0 + _NB1, 1, 2))


def _rowblock_of(j):
    return jnp.where(j < _NB0, j,
                     jnp.where(j < _NB0 + _NB1, j - _NB0, j - _NB0 - _NB1))


def kernel(features_per_scale_0, features_per_scale_1, features_per_scale_2,
           scale_embeddings, patch_embeddings):
    B = features_per_scale_0.shape[0]
    n_out = (_NTOT) * _ROWS

    BB = 8
    return pl.pallas_call(
        _body,
        grid=(_NTOT, B // BB),
        in_specs=[
            pl.BlockSpec((BB, _ROWS, _D), lambda j, b: (b, jnp.minimum(j, _NB0 - 1), 0)),
            pl.BlockSpec((BB, _ROWS, _D), lambda j, b: (b, jnp.clip(j - _NB0, 0, _NB1 - 1), 0)),
            pl.BlockSpec((BB, _ROWS, _D), lambda j, b: (b, 0, 0)),
            pl.BlockSpec((3, _D), lambda j, b: (0, 0)),
            pl.BlockSpec((1, _ROWS, _D), lambda j, b: (_scale_of(j), _rowblock_of(j), 0)),
        ],
        out_specs=pl.BlockSpec((BB, _ROWS, _D), lambda j, b: (b, j, 0)),
        out_shape=jax.ShapeDtypeStruct((B, n_out, _D), jnp.float32),
        compiler_params=pltpu.CompilerParams(
            dimension_semantics=("parallel", "parallel")),
    )(features_per_scale_0, features_per_scale_1, features_per_scale_2,
      scale_embeddings, patch_embeddings)


# speedup vs baseline: 1.8991x; 1.8991x over previous
import jax
import jax.numpy as jnp
from jax.experimental import pallas as pl
from jax.experimental.pallas import tpu as pltpu

_D = 768
_N0, _N1, _N2 = 1024, 256, 64
_NTOT = _N0 + _N1 + _N2


def _body(f0_ref, f1_ref, f2_ref, se_ref, pe0_ref, pe1_ref, pe2_ref, out_ref):
    out_ref[0, 0:_N0, :] = (
        f0_ref[0] + (se_ref[0, :][None, :] + pe0_ref[0]))
    out_ref[0, _N0:_N0 + _N1, :] = (
        f1_ref[0] + (se_ref[1, :][None, :] + pe1_ref[0]))
    out_ref[0, _N0 + _N1:_NTOT, :] = (
        f2_ref[0] + (se_ref[2, :][None, :] + pe2_ref[0]))


def kernel(features_per_scale_0, features_per_scale_1, features_per_scale_2,
           scale_embeddings, patch_embeddings):
    B = features_per_scale_0.shape[0]

    return pl.pallas_call(
        _body,
        grid=(B,),
        in_specs=[
            pl.BlockSpec((1, _N0, _D), lambda b: (b, 0, 0)),
            pl.BlockSpec((1, _N1, _D), lambda b: (b, 0, 0)),
            pl.BlockSpec((1, _N2, _D), lambda b: (b, 0, 0)),
            pl.BlockSpec((3, _D), lambda b: (0, 0)),
            pl.BlockSpec((1, _N0, _D), lambda b: (0, 0, 0)),
            pl.BlockSpec((1, _N1, _D), lambda b: (1, 0, 0)),
            pl.BlockSpec((1, _N2, _D), lambda b: (2, 0, 0)),
        ],
        out_specs=pl.BlockSpec((1, _NTOT, _D), lambda b: (b, 0, 0)),
        out_shape=jax.ShapeDtypeStruct((B, _NTOT, _D), jnp.float32),
        compiler_params=pltpu.CompilerParams(
            dimension_semantics=("parallel",)),
    )(features_per_scale_0, features_per_scale_1, features_per_scale_2,
      scale_embeddings, patch_embeddings, patch_embeddings, patch_embeddings)


# speedup vs baseline: 1.9628x; 1.0335x over previous
import jax
import jax.numpy as jnp
from jax.experimental import pallas as pl
from jax.experimental.pallas import tpu as pltpu

_D = 768
_N0, _N1, _N2 = 1024, 256, 64
_NTOT = _N0 + _N1 + _N2


_BB = 2


def _body(f0_ref, f1_ref, f2_ref, se_ref, pe0_ref, pe1_ref, pe2_ref, out_ref):
    out_ref[:, 0:_N0, :] = (
        f0_ref[...] + (se_ref[0, :][None, None, :] + pe0_ref[...]))
    out_ref[:, _N0:_N0 + _N1, :] = (
        f1_ref[...] + (se_ref[1, :][None, None, :] + pe1_ref[...]))
    out_ref[:, _N0 + _N1:_NTOT, :] = (
        f2_ref[...] + (se_ref[2, :][None, None, :] + pe2_ref[...]))


def kernel(features_per_scale_0, features_per_scale_1, features_per_scale_2,
           scale_embeddings, patch_embeddings):
    B = features_per_scale_0.shape[0]

    return pl.pallas_call(
        _body,
        grid=(B // _BB,),
        in_specs=[
            pl.BlockSpec((_BB, _N0, _D), lambda b: (b, 0, 0)),
            pl.BlockSpec((_BB, _N1, _D), lambda b: (b, 0, 0)),
            pl.BlockSpec((_BB, _N2, _D), lambda b: (b, 0, 0)),
            pl.BlockSpec((3, _D), lambda b: (0, 0)),
            pl.BlockSpec((1, _N0, _D), lambda b: (0, 0, 0)),
            pl.BlockSpec((1, _N1, _D), lambda b: (1, 0, 0)),
            pl.BlockSpec((1, _N2, _D), lambda b: (2, 0, 0)),
        ],
        out_specs=pl.BlockSpec((_BB, _NTOT, _D), lambda b: (b, 0, 0)),
        out_shape=jax.ShapeDtypeStruct((B, _NTOT, _D), jnp.float32),
        compiler_params=pltpu.CompilerParams(
            dimension_semantics=("parallel",)),
    )(features_per_scale_0, features_per_scale_1, features_per_scale_2,
      scale_embeddings, patch_embeddings, patch_embeddings, patch_embeddings)
